# baseline (speedup 1.0000x reference)
import functools

import jax
import jax.numpy as jnp
from jax import lax
from jax.experimental import pallas as pl
from jax.experimental.pallas import tpu as pltpu
from jax.experimental.pallas import tpu_sc as plsc

N = 10000
E = 320000
D_IN = 128
OUT = 16

NC = 2
NS = 16
NW = NC * NS
CHUNK = 128
NP = 10112
ESELF = E + N
NCHUNK = (-(-ESELF // (NW * CHUNK)) + 1) // 2 * 2
EPAD = NW * CHUNK * NCHUNK


def _make_sc_edge_pass(wrow, nchunk):
    rps = NP // NS
    mesh = plsc.VectorSubcoreMesh(
        core_axis_name="c", subcore_axis_name="s", num_cores=NC, num_subcores=NS
    )
    nmsg = (wrow - 16) // 16

    @functools.partial(
        pl.kernel,
        out_type=jax.ShapeDtypeStruct((NC, NP, wrow), jnp.float32),
        mesh=mesh,
        scratch_types=[
            pltpu.VMEM((nchunk, CHUNK), jnp.int32),
            pltpu.VMEM((nchunk, CHUNK), jnp.int32),
            pltpu.VMEM((2, CHUNK, wrow), jnp.float32),
            pltpu.VMEM((2, CHUNK, 16), jnp.float32),
            pltpu.VMEM((2, CHUNK, wrow), jnp.float32),
            pltpu.VMEM_SHARED((NP, wrow), jnp.float32),
            pltpu.SemaphoreType.DMA((2,)),
            pltpu.SemaphoreType.DMA((2,)),
            pltpu.SemaphoreType.DMA((2,)),
        ],
        compiler_params=pltpu.CompilerParams(use_tc_tiling_on_sc=False),
    )
    def edge_pass(srctab, aldtab, sidx_h, didx_h, zeros_h, out_h,
                  sidx_vm, didx_vm, srows_v, arows_v, accrows_v, acc_sh,
                  gs_sem, ga_sem, sc_sem):
        c = lax.axis_index("c")
        s = lax.axis_index("s")
        w = c * NS + s
        base = pl.multiple_of(s * rps, 8)
        pltpu.sync_copy(zeros_h.at[pl.ds(base, rps)],
                        acc_sh.at[pl.ds(base, rps)])
        pltpu.sync_copy(sidx_h.at[w], sidx_vm)
        pltpu.sync_copy(didx_h.at[w], didx_vm)

        def issue_gather(j, b):
            pltpu.async_copy(srctab.at[sidx_vm.at[j]], srows_v.at[b],
                             gs_sem.at[b])
            pltpu.async_copy(aldtab.at[didx_vm.at[j]], arows_v.at[b],
                             ga_sem.at[b])

        issue_gather(0, 0)
        plsc.subcore_barrier()

        def pair_body(p, carry):
            for b in range(2):
                j = 2 * p + b
                nb = 1 - b

                @pl.when(j + 1 < nchunk)
                def _():
                    issue_gather(j + 1, nb)

                pltpu.make_async_copy(srctab.at[sidx_vm.at[j]],
                                      srows_v.at[b], gs_sem.at[b]).wait()
                pltpu.make_async_copy(aldtab.at[didx_vm.at[j]],
                                      arows_v.at[b], ga_sem.at[b]).wait()

                @pl.when(j >= 2)
                def _():
                    pltpu.make_async_copy(
                        accrows_v.at[b], acc_sh.at[didx_vm.at[j - 2]],
                        sc_sem.at[b]).wait()

                @plsc.parallel_loop(0, CHUNK, unroll=8)
                def _(e):
                    ald = arows_v[b, e, :]
                    als = srows_v[b, e, pl.ds(wrow - 16, 16)]
                    t = als + ald
                    t = jnp.where(t >= 0.0, t, 0.2 * t)
                    ee = jnp.exp(t)
                    for k in range(nmsg):
                        accrows_v[b, e, pl.ds(k * 16, 16)] = (
                            srows_v[b, e, pl.ds(k * 16, 16)] * ee)
                    accrows_v[b, e, pl.ds(wrow - 16, 16)] = ee

                pltpu.async_copy(accrows_v.at[b],
                                 acc_sh.at[didx_vm.at[j]],
                                 sc_sem.at[b], add=True)
            return carry

        lax.fori_loop(0, nchunk // 2, pair_body, 0)
        for b in range(2):
            pltpu.make_async_copy(accrows_v.at[b],
                                  acc_sh.at[didx_vm.at[nchunk - 2 + b]],
                                  sc_sem.at[b]).wait()
        plsc.subcore_barrier()
        pltpu.sync_copy(acc_sh.at[pl.ds(base, rps)],
                        out_h.at[c, pl.ds(base, rps)])

    return edge_pass


def _mm_body(x_ref, w_ref, o1_ref, o2_ref, *, split):
    r = jnp.dot(x_ref[...], w_ref[...], preferred_element_type=jnp.float32)
    o1_ref[...] = r[:, :split]
    o2_ref[...] = r[:, split:]


def _project(x, wcat, split, grid=4):
    rows = NP // grid
    din = x.shape[1]
    cols = wcat.shape[1]
    return pl.pallas_call(
        functools.partial(_mm_body, split=split),
        grid=(grid,),
        in_specs=[
            pl.BlockSpec((rows, din), lambda i: (i, 0)),
            pl.BlockSpec((din, cols), lambda i: (0, 0)),
        ],
        out_specs=[
            pl.BlockSpec((rows, split), lambda i: (i, 0)),
            pl.BlockSpec((rows, cols - split), lambda i: (i, 0)),
        ],
        out_shape=[
            jax.ShapeDtypeStruct((NP, split), jnp.float32),
            jax.ShapeDtypeStruct((NP, cols - split), jnp.float32),
        ],
    )(x, wcat)


def _finalize1_body(p_ref, w_ref, b_ref, o1_ref, o2_ref):
    praw = p_ref[0] + p_ref[1]
    z = praw[:, :64] / (jnp.tile(praw[:, 64:80], (1, 4)) + 1e-16) + b_ref[...]
    z = jnp.where(z > 0.0, z, jnp.exp(jnp.minimum(z, 0.0)) - 1.0)
    r = jnp.dot(z, w_ref[...], preferred_element_type=jnp.float32)
    o1_ref[...] = r[:, :32]
    o2_ref[...] = r[:, 32:]


def _finalize1(p, wcat2, b1p, grid=8):
    rows = NP // grid
    return pl.pallas_call(
        _finalize1_body,
        grid=(grid,),
        in_specs=[
            pl.BlockSpec((NC, rows, 80), lambda i: (0, i, 0)),
            pl.BlockSpec((64, 48), lambda i: (0, 0)),
            pl.BlockSpec((1, 64), lambda i: (0, 0)),
        ],
        out_specs=[
            pl.BlockSpec((rows, 32), lambda i: (i, 0)),
            pl.BlockSpec((rows, 16), lambda i: (i, 0)),
        ],
        out_shape=[
            jax.ShapeDtypeStruct((NP, 32), jnp.float32),
            jax.ShapeDtypeStruct((NP, 16), jnp.float32),
        ],
    )(p, wcat2, b1p)


def _finalize2_body(p_ref, b_ref, o_ref):
    praw = p_ref[0] + p_ref[1]
    z = praw[:, :16] / (praw[:, 16:32] + 1e-16) + b_ref[...]
    m = jnp.max(z, axis=-1, keepdims=True)
    lse = jnp.log(jnp.sum(jnp.exp(z - m), axis=-1, keepdims=True))
    o_ref[...] = z - m - lse


def _finalize2(p2, b2, grid=8):
    rows = NP // grid
    return pl.pallas_call(
        _finalize2_body,
        grid=(grid,),
        in_specs=[
            pl.BlockSpec((NC, rows, 32), lambda i: (0, i, 0)),
            pl.BlockSpec((1, 16), lambda i: (0, 0)),
        ],
        out_specs=pl.BlockSpec((rows, 16), lambda i: (i, 0)),
        out_shape=jax.ShapeDtypeStruct((NP, 16), jnp.float32),
    )(p2, b2)


def kernel(x, edge_index, W1, a_src1, a_dst1, b1, W2, a_src2, a_dst2, b2):
    f32 = jnp.float32
    perm = jnp.arange(64).reshape(8, 8).T.reshape(-1)
    eperm = jnp.eye(64, dtype=f32)[:, perm]
    ss = (a_src1[0][:, :, None] * jnp.eye(8, dtype=f32)[:, None, :]).reshape(64, 8)
    sd = (a_dst1[0][:, :, None] * jnp.eye(8, dtype=f32)[:, None, :]).reshape(64, 8)
    m1 = jnp.concatenate([eperm, ss, ss, sd, sd], axis=1)
    wcat1 = W1 @ m1
    b1p = (b1 @ eperm).reshape(1, 64)

    q2 = jnp.concatenate(
        [jnp.eye(16, dtype=f32),
         jnp.tile(a_src2[0, 0][:, None], (1, 16)),
         jnp.tile(a_dst2[0, 0][:, None], (1, 16))], axis=1)
    wcat2 = eperm.T @ (W2 @ q2)
    b2r = b2.reshape(1, 16)

    xp = jnp.pad(x, ((0, NP - N), (0, 0)))
    loops = jnp.arange(N, dtype=jnp.int32)
    padlen = EPAD - ESELF
    trash = (N + jnp.arange(padlen, dtype=jnp.int32) % (NP - N)).astype(jnp.int32)
    srcp = jnp.concatenate([edge_index[0], loops, trash]).reshape(NW, NCHUNK, CHUNK)
    dstp = jnp.concatenate([edge_index[1], loops, trash]).reshape(NW, NCHUNK, CHUNK)
    zeros80 = jnp.zeros((NP, 80), f32)
    zeros32 = jnp.zeros((NP, 32), f32)

    srctab1, aldtab1 = _project(xp, wcat1, 80)
    p1 = _make_sc_edge_pass(80, NCHUNK)(srctab1, aldtab1, srcp, dstp, zeros80)
    srctab2, aldtab2 = _finalize1(p1, wcat2, b1p)

    p2 = _make_sc_edge_pass(32, NCHUNK)(srctab2, aldtab2, srcp, dstp, zeros32)
    out = _finalize2(p2, b2r)
    return out[:N]


# speedup vs baseline: 1.0747x; 1.0747x over previous
import functools

import jax
import jax.numpy as jnp
from jax import lax
from jax.experimental import pallas as pl
from jax.experimental.pallas import tpu as pltpu
from jax.experimental.pallas import tpu_sc as plsc

N = 10000
E = 320000
D_IN = 128
OUT = 16

NC = 2
NS = 16
NW = NC * NS
CHUNK = 128
NP = 10112
ESELF = E + N
NCHUNK = (-(-ESELF // (NW * CHUNK)) + 1) // 2 * 2
EPAD = NW * CHUNK * NCHUNK


def _make_sc_edge_pass(wrow, nchunk):
    rps = NP // NS
    mesh = plsc.VectorSubcoreMesh(
        core_axis_name="c", subcore_axis_name="s", num_cores=NC, num_subcores=NS
    )
    nmsg = (wrow - 16) // 16

    @functools.partial(
        pl.kernel,
        out_type=jax.ShapeDtypeStruct((NC, NP, wrow), jnp.float32),
        mesh=mesh,
        scratch_types=[
            pltpu.VMEM((nchunk, CHUNK), jnp.int32),
            pltpu.VMEM((nchunk, CHUNK), jnp.int32),
            pltpu.VMEM((2, CHUNK, wrow), jnp.float32),
            pltpu.VMEM((2, CHUNK, 16), jnp.float32),
            pltpu.VMEM((2, CHUNK, wrow), jnp.float32),
            pltpu.VMEM_SHARED((NP, wrow), jnp.float32),
            pltpu.SemaphoreType.DMA((2,)),
            pltpu.SemaphoreType.DMA((2,)),
            pltpu.SemaphoreType.DMA((2,)),
        ],
        compiler_params=pltpu.CompilerParams(use_tc_tiling_on_sc=False),
    )
    def edge_pass(srctab, aldtab, ei_h, zeros_h, out_h,
                  sidx_vm, didx_vm, srows_v, arows_v, accrows_v, acc_sh,
                  gs_sem, ga_sem, sc_sem):
        c = lax.axis_index("c")
        s = lax.axis_index("s")
        w = c * NS + s
        base = pl.multiple_of(s * rps, 8)
        pltpu.sync_copy(zeros_h.at[pl.ds(base, rps)],
                        acc_sh.at[pl.ds(base, rps)])
        pltpu.sync_copy(ei_h.at[0, w], sidx_vm)
        pltpu.sync_copy(ei_h.at[1, w], didx_vm)

        def issue_gather(j, b):
            pltpu.async_copy(srctab.at[sidx_vm.at[j]], srows_v.at[b],
                             gs_sem.at[b])
            pltpu.async_copy(aldtab.at[didx_vm.at[j]], arows_v.at[b],
                             ga_sem.at[b])

        issue_gather(0, 0)
        plsc.subcore_barrier()

        def pair_body(p, carry):
            for b in range(2):
                j = 2 * p + b
                nb = 1 - b

                @pl.when(j + 1 < nchunk)
                def _():
                    issue_gather(j + 1, nb)

                pltpu.make_async_copy(srctab.at[sidx_vm.at[j]],
                                      srows_v.at[b], gs_sem.at[b]).wait()
                pltpu.make_async_copy(aldtab.at[didx_vm.at[j]],
                                      arows_v.at[b], ga_sem.at[b]).wait()

                @pl.when(j >= 2)
                def _():
                    pltpu.make_async_copy(
                        accrows_v.at[b], acc_sh.at[didx_vm.at[j - 2]],
                        sc_sem.at[b]).wait()

                @plsc.parallel_loop(0, CHUNK, unroll=8)
                def _(e):
                    ald = arows_v[b, e, :]
                    als = srows_v[b, e, pl.ds(wrow - 16, 16)]
                    t = als + ald
                    t = jnp.where(t >= 0.0, t, 0.2 * t)
                    ee = jnp.exp(t)
                    for k in range(nmsg):
                        accrows_v[b, e, pl.ds(k * 16, 16)] = (
                            srows_v[b, e, pl.ds(k * 16, 16)] * ee)
                    accrows_v[b, e, pl.ds(wrow - 16, 16)] = ee

                pltpu.async_copy(accrows_v.at[b],
                                 acc_sh.at[didx_vm.at[j]],
                                 sc_sem.at[b], add=True)
            return carry

        lax.fori_loop(0, nchunk // 2, pair_body, 0)
        for b in range(2):
            pltpu.make_async_copy(accrows_v.at[b],
                                  acc_sh.at[didx_vm.at[nchunk - 2 + b]],
                                  sc_sem.at[b]).wait()
        plsc.subcore_barrier()
        pltpu.sync_copy(acc_sh.at[pl.ds(base, rps)],
                        out_h.at[c, pl.ds(base, rps)])

    return edge_pass


def _mm_body(x_ref, w_ref, m_ref, o1_ref, o2_ref, *, split):
    wc = jnp.dot(w_ref[...], m_ref[...], preferred_element_type=jnp.float32)
    r = jnp.dot(x_ref[...], wc, preferred_element_type=jnp.float32)
    o1_ref[...] = r[:, :split]
    o2_ref[...] = r[:, split:]


def _project(x, w, m, split, grid=4):
    rows = NP // grid
    din = x.shape[1]
    k = w.shape[1]
    cols = m.shape[1]
    return pl.pallas_call(
        functools.partial(_mm_body, split=split),
        grid=(grid,),
        in_specs=[
            pl.BlockSpec((rows, din), lambda i: (i, 0)),
            pl.BlockSpec((din, k), lambda i: (0, 0)),
            pl.BlockSpec((k, cols), lambda i: (0, 0)),
        ],
        out_specs=[
            pl.BlockSpec((rows, split), lambda i: (i, 0)),
            pl.BlockSpec((rows, cols - split), lambda i: (i, 0)),
        ],
        out_shape=[
            jax.ShapeDtypeStruct((NP, split), jnp.float32),
            jax.ShapeDtypeStruct((NP, cols - split), jnp.float32),
        ],
    )(x, w, m)


def _finalize1_body(p_ref, w_ref, q_ref, ep_ref, b_ref, o1_ref, o2_ref):
    w2q = jnp.dot(w_ref[...], q_ref[...], preferred_element_type=jnp.float32)
    wc = lax.dot_general(ep_ref[...], w2q, (((0,), (0,)), ((), ())),
                         preferred_element_type=jnp.float32)
    bp = jnp.dot(b_ref[...], ep_ref[...], preferred_element_type=jnp.float32)
    praw = p_ref[0] + p_ref[1]
    z = praw[:, :64] / (jnp.tile(praw[:, 64:80], (1, 4)) + 1e-16) + bp
    z = jnp.where(z > 0.0, z, jnp.exp(jnp.minimum(z, 0.0)) - 1.0)
    r = jnp.dot(z, wc, preferred_element_type=jnp.float32)
    o1_ref[...] = r[:, :32]
    o2_ref[...] = r[:, 32:]


def _finalize1(p, W2, q2, epermT, b1, grid=8):
    rows = NP // grid
    return pl.pallas_call(
        _finalize1_body,
        grid=(grid,),
        in_specs=[
            pl.BlockSpec((NC, rows, 80), lambda i: (0, i, 0)),
            pl.BlockSpec((64, 16), lambda i: (0, 0)),
            pl.BlockSpec((16, 48), lambda i: (0, 0)),
            pl.BlockSpec((64, 64), lambda i: (0, 0)),
            pl.BlockSpec((1, 64), lambda i: (0, 0)),
        ],
        out_specs=[
            pl.BlockSpec((rows, 32), lambda i: (i, 0)),
            pl.BlockSpec((rows, 16), lambda i: (i, 0)),
        ],
        out_shape=[
            jax.ShapeDtypeStruct((NP, 32), jnp.float32),
            jax.ShapeDtypeStruct((NP, 16), jnp.float32),
        ],
    )(p, W2, q2, epermT, b1)


def _finalize2_body(p_ref, b_ref, o_ref):
    praw = p_ref[0] + p_ref[1]
    z = praw[:, :16] / (praw[:, 16:32] + 1e-16) + b_ref[...]
    m = jnp.max(z, axis=-1, keepdims=True)
    lse = jnp.log(jnp.sum(jnp.exp(z - m), axis=-1, keepdims=True))
    o_ref[...] = z - m - lse


def _finalize2(p2, b2, grid=8):
    rows = NP // grid
    return pl.pallas_call(
        _finalize2_body,
        grid=(grid,),
        in_specs=[
            pl.BlockSpec((NC, rows, 32), lambda i: (0, i, 0)),
            pl.BlockSpec((1, 16), lambda i: (0, 0)),
        ],
        out_specs=pl.BlockSpec((rows, 16), lambda i: (i, 0)),
        out_shape=jax.ShapeDtypeStruct((NP, 16), jnp.float32),
    )(p2, b2)


def kernel(x, edge_index, W1, a_src1, a_dst1, b1, W2, a_src2, a_dst2, b2):
    f32 = jnp.float32
    perm = jnp.arange(64).reshape(8, 8).T.reshape(-1)
    eperm = jnp.eye(64, dtype=f32)[:, perm]
    ss = (a_src1[0][:, :, None] * jnp.eye(8, dtype=f32)[:, None, :]).reshape(64, 8)
    sd = (a_dst1[0][:, :, None] * jnp.eye(8, dtype=f32)[:, None, :]).reshape(64, 8)
    m1 = jnp.concatenate([eperm, ss, ss, sd, sd], axis=1)

    q2 = jnp.concatenate(
        [jnp.eye(16, dtype=f32),
         jnp.tile(a_src2[0, 0][:, None], (1, 16)),
         jnp.tile(a_dst2[0, 0][:, None], (1, 16))], axis=1)
    b1r = b1.reshape(1, 64)
    b2r = b2.reshape(1, 16)

    xp = jnp.pad(x, ((0, NP - N), (0, 0)))
    loops = jnp.arange(N, dtype=jnp.int32)
    padlen = EPAD - ESELF
    trash = (N + jnp.arange(padlen, dtype=jnp.int32) % (NP - N)).astype(jnp.int32)
    extra = jnp.concatenate([loops, trash])
    ei_all = jnp.concatenate(
        [edge_index, jnp.broadcast_to(extra, (2, extra.size))],
        axis=1).reshape(2, NW, NCHUNK, CHUNK)
    zeros80 = jnp.zeros((NP, 80), f32)
    zeros32 = jnp.zeros((NP, 32), f32)

    srctab1, aldtab1 = _project(xp, W1, m1, 80)
    p1 = _make_sc_edge_pass(80, NCHUNK)(srctab1, aldtab1, ei_all, zeros80)
    srctab2, aldtab2 = _finalize1(p1, W2, q2, eperm, b1r)

    p2 = _make_sc_edge_pass(32, NCHUNK)(srctab2, aldtab2, ei_all, zeros32)
    out = _finalize2(p2, b2r)
    return out[:N]


# speedup vs baseline: 1.1116x; 1.0343x over previous
import functools

import jax
import jax.numpy as jnp
from jax import lax
from jax.experimental import pallas as pl
from jax.experimental.pallas import tpu as pltpu
from jax.experimental.pallas import tpu_sc as plsc

N = 10000
E = 320000
D_IN = 128
OUT = 16

NC = 2
NS = 16
NW = NC * NS
CHUNK = 128
NP = 10112
ESELF = E + N
NCHUNK = (-(-ESELF // (NW * CHUNK)) + 1) // 2 * 2
EPAD = NW * CHUNK * NCHUNK
ECHUNKS = E // CHUNK
XCHUNKS = NW * NCHUNK - ECHUNKS
WB = ECHUNKS // NCHUNK
WB_REAL = ECHUNKS - WB * NCHUNK


def _make_sc_edge_pass(wrow, nchunk):
    rps = NP // NS
    mesh = plsc.VectorSubcoreMesh(
        core_axis_name="c", subcore_axis_name="s", num_cores=NC, num_subcores=NS
    )
    nmsg = (wrow - 16) // 16

    @functools.partial(
        pl.kernel,
        out_type=jax.ShapeDtypeStruct((NC, NP, wrow), jnp.float32),
        mesh=mesh,
        scratch_types=[
            pltpu.VMEM((nchunk, CHUNK), jnp.int32),
            pltpu.VMEM((nchunk, CHUNK), jnp.int32),
            pltpu.VMEM((2, CHUNK, wrow), jnp.float32),
            pltpu.VMEM((2, CHUNK, 16), jnp.float32),
            pltpu.VMEM((2, CHUNK, wrow), jnp.float32),
            pltpu.VMEM_SHARED((NP, wrow), jnp.float32),
            pltpu.SemaphoreType.DMA((2,)),
            pltpu.SemaphoreType.DMA((2,)),
            pltpu.SemaphoreType.DMA((2,)),
        ],
        compiler_params=pltpu.CompilerParams(use_tc_tiling_on_sc=False),
    )
    def edge_pass(srctab, aldtab, ei_h, ex_h, out_h,
                  sidx_vm, didx_vm, srows_v, arows_v, accrows_v, acc_sh,
                  gs_sem, ga_sem, sc_sem):
        c = lax.axis_index("c")
        s = lax.axis_index("s")
        w = c * NS + s
        base = pl.multiple_of(s * rps, 8)

        @plsc.parallel_loop(0, CHUNK, unroll=8)
        def _(e):
            for k in range(wrow // 16):
                accrows_v[0, e, pl.ds(k * 16, 16)] = jnp.zeros((16,), jnp.float32)

        for k in range(rps // CHUNK):
            pltpu.sync_copy(accrows_v.at[0],
                            acc_sh.at[pl.ds(base + k * CHUNK, CHUNK)])
        rem = rps % CHUNK
        if rem:
            pltpu.sync_copy(
                accrows_v.at[0, pl.ds(0, rem)],
                acc_sh.at[pl.ds(base + (rps // CHUNK) * CHUNK, rem)])

        wb_extra = nchunk - WB_REAL
        for r in range(2):
            idx_vm = sidx_vm if r == 0 else didx_vm

            @pl.when(w < WB)
            def _():
                pltpu.sync_copy(ei_h.at[r, pl.ds(w * nchunk, nchunk)], idx_vm)

            @pl.when(w == WB)
            def _():
                pltpu.sync_copy(ei_h.at[r, pl.ds(WB * nchunk, WB_REAL)],
                                idx_vm.at[pl.ds(0, WB_REAL)])
                pltpu.sync_copy(ex_h.at[r, pl.ds(0, wb_extra)],
                                idx_vm.at[pl.ds(WB_REAL, wb_extra)])

            @pl.when(w > WB)
            def _():
                pltpu.sync_copy(
                    ex_h.at[r, pl.ds(wb_extra + (w - WB - 1) * nchunk, nchunk)],
                    idx_vm)

        def issue_gather(j, b):
            pltpu.async_copy(srctab.at[sidx_vm.at[j]], srows_v.at[b],
                             gs_sem.at[b])
            pltpu.async_copy(aldtab.at[didx_vm.at[j]], arows_v.at[b],
                             ga_sem.at[b])

        issue_gather(0, 0)
        plsc.subcore_barrier()

        def pair_body(p, carry):
            for b in range(2):
                j = 2 * p + b
                nb = 1 - b

                @pl.when(j + 1 < nchunk)
                def _():
                    issue_gather(j + 1, nb)

                pltpu.make_async_copy(srctab.at[sidx_vm.at[j]],
                                      srows_v.at[b], gs_sem.at[b]).wait()
                pltpu.make_async_copy(aldtab.at[didx_vm.at[j]],
                                      arows_v.at[b], ga_sem.at[b]).wait()

                @pl.when(j >= 2)
                def _():
                    pltpu.make_async_copy(
                        accrows_v.at[b], acc_sh.at[didx_vm.at[j - 2]],
                        sc_sem.at[b]).wait()

                @plsc.parallel_loop(0, CHUNK, unroll=8)
                def _(e):
                    ald = arows_v[b, e, :]
                    als = srows_v[b, e, pl.ds(wrow - 16, 16)]
                    t = als + ald
                    t = jnp.where(t >= 0.0, t, 0.2 * t)
                    ee = jnp.exp(t)
                    for k in range(nmsg):
                        accrows_v[b, e, pl.ds(k * 16, 16)] = (
                            srows_v[b, e, pl.ds(k * 16, 16)] * ee)
                    accrows_v[b, e, pl.ds(wrow - 16, 16)] = ee

                pltpu.async_copy(accrows_v.at[b],
                                 acc_sh.at[didx_vm.at[j]],
                                 sc_sem.at[b], add=True)
            return carry

        lax.fori_loop(0, nchunk // 2, pair_body, 0)
        for b in range(2):
            pltpu.make_async_copy(accrows_v.at[b],
                                  acc_sh.at[didx_vm.at[nchunk - 2 + b]],
                                  sc_sem.at[b]).wait()
        plsc.subcore_barrier()
        pltpu.sync_copy(acc_sh.at[pl.ds(base, rps)],
                        out_h.at[c, pl.ds(base, rps)])

    return edge_pass


def _mm_body(x_ref, w_ref, m_ref, o1_ref, o2_ref, *, split):
    wc = jnp.dot(w_ref[...], m_ref[...], preferred_element_type=jnp.float32)
    r = jnp.dot(x_ref[...], wc, preferred_element_type=jnp.float32)
    o1_ref[...] = r[:, :split]
    o2_ref[...] = r[:, split:]


def _project(x, w, m, split, grid=10):
    rows = N // grid
    din = x.shape[1]
    k = w.shape[1]
    cols = m.shape[1]
    return pl.pallas_call(
        functools.partial(_mm_body, split=split),
        grid=(grid,),
        in_specs=[
            pl.BlockSpec((rows, din), lambda i: (i, 0)),
            pl.BlockSpec((din, k), lambda i: (0, 0)),
            pl.BlockSpec((k, cols), lambda i: (0, 0)),
        ],
        out_specs=[
            pl.BlockSpec((rows, split), lambda i: (i, 0)),
            pl.BlockSpec((rows, cols - split), lambda i: (i, 0)),
        ],
        out_shape=[
            jax.ShapeDtypeStruct((NP, split), jnp.float32),
            jax.ShapeDtypeStruct((NP, cols - split), jnp.float32),
        ],
    )(x, w, m)


def _finalize1_body(p_ref, w_ref, q_ref, ep_ref, b_ref, o1_ref, o2_ref):
    w2q = jnp.dot(w_ref[...], q_ref[...], preferred_element_type=jnp.float32)
    wc = lax.dot_general(ep_ref[...], w2q, (((0,), (0,)), ((), ())),
                         preferred_element_type=jnp.float32)
    bp = jnp.dot(b_ref[...], ep_ref[...], preferred_element_type=jnp.float32)
    praw = p_ref[0] + p_ref[1]
    z = praw[:, :64] / (jnp.tile(praw[:, 64:80], (1, 4)) + 1e-16) + bp
    z = jnp.where(z > 0.0, z, jnp.exp(jnp.minimum(z, 0.0)) - 1.0)
    r = jnp.dot(z, wc, preferred_element_type=jnp.float32)
    o1_ref[...] = r[:, :32]
    o2_ref[...] = r[:, 32:]


def _finalize1(p, W2, q2, epermT, b1, grid=8):
    rows = NP // grid
    return pl.pallas_call(
        _finalize1_body,
        grid=(grid,),
        in_specs=[
            pl.BlockSpec((NC, rows, 80), lambda i: (0, i, 0)),
            pl.BlockSpec((64, 16), lambda i: (0, 0)),
            pl.BlockSpec((16, 48), lambda i: (0, 0)),
            pl.BlockSpec((64, 64), lambda i: (0, 0)),
            pl.BlockSpec((1, 64), lambda i: (0, 0)),
        ],
        out_specs=[
            pl.BlockSpec((rows, 32), lambda i: (i, 0)),
            pl.BlockSpec((rows, 16), lambda i: (i, 0)),
        ],
        out_shape=[
            jax.ShapeDtypeStruct((NP, 32), jnp.float32),
            jax.ShapeDtypeStruct((NP, 16), jnp.float32),
        ],
    )(p, W2, q2, epermT, b1)


def _finalize2_body(p_ref, b_ref, o_ref):
    praw = p_ref[0] + p_ref[1]
    z = praw[:, :16] / (praw[:, 16:32] + 1e-16) + b_ref[...]
    m = jnp.max(z, axis=-1, keepdims=True)
    lse = jnp.log(jnp.sum(jnp.exp(z - m), axis=-1, keepdims=True))
    o_ref[...] = z - m - lse


def _finalize2(p2, b2, grid=10):
    rows = N // grid
    return pl.pallas_call(
        _finalize2_body,
        grid=(grid,),
        in_specs=[
            pl.BlockSpec((NC, rows, 32), lambda i: (0, i, 0)),
            pl.BlockSpec((1, 16), lambda i: (0, 0)),
        ],
        out_specs=pl.BlockSpec((rows, 16), lambda i: (i, 0)),
        out_shape=jax.ShapeDtypeStruct((N, 16), jnp.float32),
    )(p2, b2)


def kernel(x, edge_index, W1, a_src1, a_dst1, b1, W2, a_src2, a_dst2, b2):
    f32 = jnp.float32
    perm = jnp.arange(64).reshape(8, 8).T.reshape(-1)
    eperm = jnp.eye(64, dtype=f32)[:, perm]
    ss = (a_src1[0][:, :, None] * jnp.eye(8, dtype=f32)[:, None, :]).reshape(64, 8)
    sd = (a_dst1[0][:, :, None] * jnp.eye(8, dtype=f32)[:, None, :]).reshape(64, 8)
    m1 = jnp.concatenate([eperm, ss, ss, sd, sd], axis=1)

    q2 = jnp.concatenate(
        [jnp.eye(16, dtype=f32),
         jnp.tile(a_src2[0, 0][:, None], (1, 16)),
         jnp.tile(a_dst2[0, 0][:, None], (1, 16))], axis=1)
    b1r = b1.reshape(1, 64)
    b2r = b2.reshape(1, 16)

    loops = jnp.arange(N, dtype=jnp.int32)
    padlen = EPAD - ESELF
    trash = (N + jnp.arange(padlen, dtype=jnp.int32) % (NP - N)).astype(jnp.int32)
    extra = jnp.concatenate([loops, trash])
    ei3 = edge_index.reshape(2, ECHUNKS, CHUNK)
    ex3 = jnp.broadcast_to(extra, (2, extra.size)).reshape(2, XCHUNKS, CHUNK)

    srctab1, aldtab1 = _project(x, W1, m1, 80)
    p1 = _make_sc_edge_pass(80, NCHUNK)(srctab1, aldtab1, ei3, ex3)
    srctab2, aldtab2 = _finalize1(p1, W2, q2, eperm, b1r)

    p2 = _make_sc_edge_pass(32, NCHUNK)(srctab2, aldtab2, ei3, ex3)
    return _finalize2(p2, b2r)
